# baseline (speedup 1.0000x reference)
import jax
import jax.numpy as jnp
from jax.experimental import pallas as pl
from jax.experimental.pallas import tpu as pltpu

B, L, O, DM = 8, 48, 100, 128
OP = 112
LT = 8

_f32 = jnp.float32
_bf16 = jnp.bfloat16


def _main_step(x_ref, dis_ref, w1_ref, b1_ref, w2_ref, b2_ref,
               out_ref, disw_ref, w1s_ref, x16_ref, xt16_ref, tdn_ref):
    @pl.when(pl.program_id(0) == 0)
    def _init():
        dis = dis_ref[...]
        sd = jnp.sqrt(dis)
        dw = jnp.where(dis <= 2.0, sd, 0.0) / jnp.sum(sd, axis=1,
                                                      keepdims=True)
        disw_ref[...] = jnp.zeros((OP, OP), _bf16)
        disw_ref[:O, :O] = dw.astype(_bf16)
        w1 = w1_ref[...]
        w1s_ref[...] = jnp.zeros((2 * OP, DM), _bf16)
        w1s_ref[:O] = w1[:O].astype(_bf16)
        w1s_ref[O:O + 1] = b1_ref[...].astype(_bf16)
        w1s_ref[OP:OP + O] = w1[O:].astype(_bf16)
        x16_ref[...] = jnp.zeros((LT, OP, OP), _bf16)
        x16_ref[:, :, O:O + 1] = jnp.ones((LT, OP, 1), _bf16)
        xt16_ref[...] = jnp.zeros((LT, OP, OP), _bf16)
        tdn_ref[...] = jnp.zeros((LT, OP), _f32)

    w1s = w1s_ref[...]
    w2 = w2_ref[...].astype(_bf16)
    b2v = b2_ref[0]
    disw = disw_ref[...]

    x3 = x_ref[...]
    xts = [x3[t].T for t in range(LT)]
    x16_ref[:, :O, :O] = x3.astype(_bf16)
    for t in range(LT):
        xt16_ref[t, :O, :O] = xts[t].astype(_bf16)

    td = jnp.sum(x3 + jnp.stack(xts), axis=1)
    tdn_ref[:, :O] = td / jnp.sum(td, axis=1, keepdims=True)

    x16_3 = x16_ref[...]
    xt16_3 = xt16_ref[...]
    xall = x16_3.reshape(LT * OP, OP)
    xtall = xt16_3.reshape(LT * OP, OP)
    yall = (jnp.dot(xall, w1s[:OP], preferred_element_type=_f32)
            + jnp.dot(xtall, w1s[OP:], preferred_element_type=_f32))
    y16 = yall.astype(_bf16)

    ycat = jnp.concatenate([y16[t * OP:(t + 1) * OP] for t in range(LT)],
                           axis=1)
    fall = jnp.dot(disw, ycat, preferred_element_type=_f32)

    tdnv = tdn_ref[...]
    geo_in = []
    sem_in = []
    for t in range(LT):
        sl = slice(t * OP, (t + 1) * OP)
        tdn16 = tdnv[t:t + 1].astype(_bf16)
        deg_w = jnp.where((x16_3[t] > 0) | (xt16_3[t] > 0),
                          jnp.broadcast_to(tdn16, (OP, OP)),
                          _bf16(0))
        bt = jnp.dot(deg_w, y16[sl], preferred_element_type=_f32)
        yt = yall[sl]
        geo_in.append((yt + fall[:, t * DM:(t + 1) * DM]).astype(_bf16))
        sem_in.append((yt + bt).astype(_bf16))

    geo_all = jnp.dot(jnp.concatenate(geo_in, axis=0), w2,
                      preferred_element_type=_f32) + b2v
    sem_all = jnp.dot(jnp.concatenate(sem_in, axis=0), w2,
                      preferred_element_type=_f32) + b2v
    for t in range(LT):
        out_ref[t] = jnp.concatenate([geo_all[t * OP:t * OP + O],
                                      sem_all[t * OP:t * OP + O]], axis=-1)


def kernel(X, dis_matrix, W1, b1, W2, b2):
    Bx, Lx, Ox, _ = X.shape
    n = Bx * Lx
    Xr = X.reshape(n, Ox, Ox)

    out = pl.pallas_call(
        _main_step,
        grid=(n // LT,),
        in_specs=[
            pl.BlockSpec((LT, Ox, Ox), lambda i: (i, 0, 0)),
            pl.BlockSpec((Ox, Ox), lambda i: (0, 0)),
            pl.BlockSpec((2 * Ox, DM), lambda i: (0, 0)),
            pl.BlockSpec((1, DM), lambda i: (0, 0)),
            pl.BlockSpec((DM, DM), lambda i: (0, 0)),
            pl.BlockSpec((1, DM), lambda i: (0, 0)),
        ],
        out_specs=pl.BlockSpec((LT, Ox, 2 * DM), lambda i: (i, 0, 0)),
        out_shape=jax.ShapeDtypeStruct((n, Ox, 2 * DM), _f32),
        scratch_shapes=[pltpu.VMEM((OP, OP), _bf16),
                        pltpu.VMEM((2 * OP, DM), _bf16),
                        pltpu.VMEM((LT, OP, OP), _bf16),
                        pltpu.VMEM((LT, OP, OP), _bf16),
                        pltpu.VMEM((LT, OP), _f32)],
    )(Xr, dis_matrix, W1, b1.reshape(1, DM), W2, b2.reshape(1, DM))
    return out.reshape(Bx, Lx, Ox, 2 * DM)


# speedup vs baseline: 2.1792x; 2.1792x over previous
import jax
import jax.numpy as jnp
from jax.experimental import pallas as pl
from jax.experimental.pallas import tpu as pltpu

B, L, O, DM = 8, 48, 100, 128
OP = 112
LT = 8
NG = L // LT
NS = B * NG

_f32 = jnp.float32
_bf16 = jnp.bfloat16


def _main_step(xv_ref, dis_ref, w1_ref, b1_ref, w2_ref, b2_ref,
               out_ref, disw_ref, w1s_ref, x16_ref, xt16_ref, tdn_ref,
               xin_ref, sout_ref, in_sem, out_sem):
    b = pl.program_id(0)
    g = pl.program_id(1)
    s = b * NG + g
    slot = jax.lax.rem(s, 2)
    nslot = 1 - slot

    def in_copy(sl, bb, gg, t):
        return pltpu.make_async_copy(
            xv_ref.at[bb, :, gg * LT + t, :], xin_ref.at[sl, t],
            in_sem.at[sl, t])

    def out_copy(sl, bb, gg, t):
        return pltpu.make_async_copy(
            sout_ref.at[sl, t], out_ref.at[bb, :, gg * LT + t, :],
            out_sem.at[sl, t])

    @pl.when(s == 0)
    def _first():
        for t in range(LT):
            in_copy(0, 0, 0, t).start()

    @pl.when(s < NS - 1)
    def _prefetch():
        nb = jnp.where(g == NG - 1, b + 1, b)
        ng = jnp.where(g == NG - 1, 0, g + 1)
        for t in range(LT):
            in_copy(nslot, nb, ng, t).start()

    @pl.when(pl.program_id(0) + pl.program_id(1) == 0)
    def _init():
        dis = dis_ref[...]
        sd = jnp.sqrt(dis)
        dw = jnp.where(dis <= 2.0, sd, 0.0) / jnp.sum(sd, axis=1,
                                                      keepdims=True)
        disw_ref[...] = jnp.zeros((OP, OP), _bf16)
        disw_ref[:O, :O] = dw.astype(_bf16)
        w1 = w1_ref[...]
        w1s_ref[...] = jnp.zeros((2 * OP, DM), _bf16)
        w1s_ref[:O] = w1[:O].astype(_bf16)
        w1s_ref[O:O + 1] = b1_ref[...].astype(_bf16)
        w1s_ref[OP:OP + O] = w1[O:].astype(_bf16)
        x16_ref[...] = jnp.zeros((LT, OP, OP), _bf16)
        x16_ref[:, :, O:O + 1] = jnp.ones((LT, OP, 1), _bf16)
        xt16_ref[...] = jnp.zeros((LT, OP, OP), _bf16)
        tdn_ref[...] = jnp.zeros((LT, OP), _f32)

    for t in range(LT):
        in_copy(slot, b, g, t).wait()

    @pl.when(s >= 2)
    def _drain_prev():
        pb = jnp.where(g == NG - 1, b, jnp.where(g == 0, b - 1, b))
        ps = s - 2
        pbb = ps // NG
        pgg = jax.lax.rem(ps, NG)
        for t in range(LT):
            out_copy(slot, pbb, pgg, t).wait()

    w1s = w1s_ref[...]
    w2 = w2_ref[...].astype(_bf16)
    b2v = b2_ref[0]
    disw = disw_ref[...]

    x3 = xin_ref[slot]
    xts = [x3[t].T for t in range(LT)]
    x16_ref[:, :O, :O] = x3.astype(_bf16)
    for t in range(LT):
        xt16_ref[t, :O, :O] = xts[t].astype(_bf16)

    td = jnp.sum(x3 + jnp.stack(xts), axis=1)
    tdn_ref[:, :O] = td / jnp.sum(td, axis=1, keepdims=True)

    x16_3 = x16_ref[...]
    xt16_3 = xt16_ref[...]
    xall = x16_3.reshape(LT * OP, OP)
    xtall = xt16_3.reshape(LT * OP, OP)
    yall = (jnp.dot(xall, w1s[:OP], preferred_element_type=_f32)
            + jnp.dot(xtall, w1s[OP:], preferred_element_type=_f32))
    y16 = yall.astype(_bf16)

    ycat = jnp.concatenate([y16[t * OP:(t + 1) * OP] for t in range(LT)],
                           axis=1)
    fall = jnp.dot(disw, ycat, preferred_element_type=_f32)

    tdnv = tdn_ref[...]
    geo_in = []
    sem_in = []
    for t in range(LT):
        sl = slice(t * OP, (t + 1) * OP)
        tdn16 = tdnv[t:t + 1].astype(_bf16)
        deg_w = jnp.where((x16_3[t] > 0) | (xt16_3[t] > 0),
                          jnp.broadcast_to(tdn16, (OP, OP)),
                          _bf16(0))
        bt = jnp.dot(deg_w, y16[sl], preferred_element_type=_f32)
        yt = yall[sl]
        geo_in.append((yt + fall[:, t * DM:(t + 1) * DM]).astype(_bf16))
        sem_in.append((yt + bt).astype(_bf16))

    geo_all = jnp.dot(jnp.concatenate(geo_in, axis=0), w2,
                      preferred_element_type=_f32) + b2v
    sem_all = jnp.dot(jnp.concatenate(sem_in, axis=0), w2,
                      preferred_element_type=_f32) + b2v
    for t in range(LT):
        sout_ref[slot, t] = jnp.concatenate(
            [geo_all[t * OP:t * OP + O], sem_all[t * OP:t * OP + O]],
            axis=-1)
        out_copy(slot, b, g, t).start()

    @pl.when(s == NS - 1)
    def _drain_last():
        ps = s - 1
        pbb = ps // NG
        pgg = jax.lax.rem(ps, NG)
        for t in range(LT):
            out_copy(nslot, pbb, pgg, t).wait()
        for t in range(LT):
            out_copy(slot, b, g, t).wait()


def kernel(X, dis_matrix, W1, b1, W2, b2):
    Bx, Lx, Ox, _ = X.shape
    Xv = jnp.transpose(X, (0, 2, 1, 3))

    outv = pl.pallas_call(
        _main_step,
        grid=(Bx, Lx // LT),
        in_specs=[
            pl.BlockSpec(memory_space=pl.ANY),
            pl.BlockSpec((Ox, Ox), lambda b, g: (0, 0)),
            pl.BlockSpec((2 * Ox, DM), lambda b, g: (0, 0)),
            pl.BlockSpec((1, DM), lambda b, g: (0, 0)),
            pl.BlockSpec((DM, DM), lambda b, g: (0, 0)),
            pl.BlockSpec((1, DM), lambda b, g: (0, 0)),
        ],
        out_specs=pl.BlockSpec(memory_space=pl.ANY),
        out_shape=jax.ShapeDtypeStruct((Bx, Ox, Lx, 2 * DM), _f32),
        scratch_shapes=[pltpu.VMEM((OP, OP), _bf16),
                        pltpu.VMEM((2 * OP, DM), _bf16),
                        pltpu.VMEM((LT, OP, OP), _bf16),
                        pltpu.VMEM((LT, OP, OP), _bf16),
                        pltpu.VMEM((LT, OP), _f32),
                        pltpu.VMEM((2, LT, Ox, Ox), _f32),
                        pltpu.VMEM((2, LT, Ox, 2 * DM), _f32),
                        pltpu.SemaphoreType.DMA((2, LT)),
                        pltpu.SemaphoreType.DMA((2, LT))],
    )(Xv, dis_matrix, W1, b1.reshape(1, DM), W2, b2.reshape(1, DM))
    return jnp.transpose(outv, (0, 2, 1, 3))


# speedup vs baseline: 2.9549x; 1.3560x over previous
import jax
import jax.numpy as jnp
from jax.experimental import pallas as pl
from jax.experimental.pallas import tpu as pltpu

B, L, O, DM = 8, 48, 100, 128
OP = 112
LT = 16
NG = L // LT
NS = B * NG

_f32 = jnp.float32
_bf16 = jnp.bfloat16


def _main_step(xv_ref, dis_ref, w1_ref, b1_ref, w2_ref, b2_ref,
               out_ref, disw_ref, w1s_ref, x16_ref, xt16_ref, tdn_ref,
               xin_ref, sout_ref, in_sem, out_sem):
    b = pl.program_id(0)
    g = pl.program_id(1)
    s = b * NG + g
    slot = jax.lax.rem(s, 2)
    nslot = 1 - slot

    def in_copy(sl, bb, gg, t):
        return pltpu.make_async_copy(
            xv_ref.at[bb, :, gg * LT + t, :], xin_ref.at[sl, t],
            in_sem.at[sl, t])

    def out_copy(sl, bb, gg, t):
        return pltpu.make_async_copy(
            sout_ref.at[sl, t], out_ref.at[bb, :, gg * LT + t, :],
            out_sem.at[sl, t])

    @pl.when(s == 0)
    def _first():
        for t in range(LT):
            in_copy(0, 0, 0, t).start()

    @pl.when(s < NS - 1)
    def _prefetch():
        nb = jnp.where(g == NG - 1, b + 1, b)
        ng = jnp.where(g == NG - 1, 0, g + 1)
        for t in range(LT):
            in_copy(nslot, nb, ng, t).start()

    @pl.when(pl.program_id(0) + pl.program_id(1) == 0)
    def _init():
        dis = dis_ref[...]
        sd = jnp.sqrt(dis)
        dw = jnp.where(dis <= 2.0, sd, 0.0) / jnp.sum(sd, axis=1,
                                                      keepdims=True)
        disw_ref[...] = jnp.zeros((OP, OP), _bf16)
        disw_ref[:O, :O] = dw.astype(_bf16)
        w1 = w1_ref[...]
        w1s_ref[...] = jnp.zeros((2 * OP, DM), _bf16)
        w1s_ref[:O] = w1[:O].astype(_bf16)
        w1s_ref[O:O + 1] = b1_ref[...].astype(_bf16)
        w1s_ref[OP:OP + O] = w1[O:].astype(_bf16)
        x16_ref[...] = jnp.zeros((LT, OP, OP), _bf16)
        x16_ref[:, :, O:O + 1] = jnp.ones((LT, OP, 1), _bf16)
        xt16_ref[...] = jnp.zeros((LT, OP, OP), _bf16)
        tdn_ref[...] = jnp.zeros((LT, OP), _f32)

    for t in range(LT):
        in_copy(slot, b, g, t).wait()

    @pl.when(s >= 2)
    def _drain_prev():
        pb = jnp.where(g == NG - 1, b, jnp.where(g == 0, b - 1, b))
        ps = s - 2
        pbb = ps // NG
        pgg = jax.lax.rem(ps, NG)
        for t in range(LT):
            out_copy(slot, pbb, pgg, t).wait()

    w1s = w1s_ref[...]
    w2 = w2_ref[...].astype(_bf16)
    b2v = b2_ref[0]
    disw = disw_ref[...]

    x3 = xin_ref[slot]
    xts = [x3[t].T for t in range(LT)]
    x16_ref[:, :O, :O] = x3.astype(_bf16)
    for t in range(LT):
        xt16_ref[t, :O, :O] = xts[t].astype(_bf16)

    td = jnp.sum(x3 + jnp.stack(xts), axis=1)
    tdn_ref[:, :O] = td / jnp.sum(td, axis=1, keepdims=True)

    x16_3 = x16_ref[...]
    xt16_3 = xt16_ref[...]
    xall = x16_3.reshape(LT * OP, OP)
    xtall = xt16_3.reshape(LT * OP, OP)
    yall = (jnp.dot(xall, w1s[:OP], preferred_element_type=_f32)
            + jnp.dot(xtall, w1s[OP:], preferred_element_type=_f32))
    y16 = yall.astype(_bf16)

    ycat = jnp.concatenate([y16[t * OP:(t + 1) * OP] for t in range(LT)],
                           axis=1)
    fall = jnp.dot(disw, ycat, preferred_element_type=_f32)

    tdnv = tdn_ref[...]
    geo_in = []
    sem_in = []
    for t in range(LT):
        sl = slice(t * OP, (t + 1) * OP)
        tdn16 = tdnv[t:t + 1].astype(_bf16)
        deg_w = jnp.where((x16_3[t] > 0) | (xt16_3[t] > 0),
                          jnp.broadcast_to(tdn16, (OP, OP)),
                          _bf16(0))
        bt = jnp.dot(deg_w, y16[sl], preferred_element_type=_f32)
        yt = yall[sl]
        geo_in.append((yt + fall[:, t * DM:(t + 1) * DM]).astype(_bf16))
        sem_in.append((yt + bt).astype(_bf16))

    geo_all = jnp.dot(jnp.concatenate(geo_in, axis=0), w2,
                      preferred_element_type=_f32) + b2v
    sem_all = jnp.dot(jnp.concatenate(sem_in, axis=0), w2,
                      preferred_element_type=_f32) + b2v
    for t in range(LT):
        sout_ref[slot, t] = jnp.concatenate(
            [geo_all[t * OP:t * OP + O], sem_all[t * OP:t * OP + O]],
            axis=-1)
        out_copy(slot, b, g, t).start()

    @pl.when(s == NS - 1)
    def _drain_last():
        ps = s - 1
        pbb = ps // NG
        pgg = jax.lax.rem(ps, NG)
        for t in range(LT):
            out_copy(nslot, pbb, pgg, t).wait()
        for t in range(LT):
            out_copy(slot, b, g, t).wait()


def kernel(X, dis_matrix, W1, b1, W2, b2):
    Bx, Lx, Ox, _ = X.shape
    Xv = jnp.transpose(X, (0, 2, 1, 3))

    outv = pl.pallas_call(
        _main_step,
        grid=(Bx, Lx // LT),
        in_specs=[
            pl.BlockSpec(memory_space=pl.ANY),
            pl.BlockSpec((Ox, Ox), lambda b, g: (0, 0)),
            pl.BlockSpec((2 * Ox, DM), lambda b, g: (0, 0)),
            pl.BlockSpec((1, DM), lambda b, g: (0, 0)),
            pl.BlockSpec((DM, DM), lambda b, g: (0, 0)),
            pl.BlockSpec((1, DM), lambda b, g: (0, 0)),
        ],
        out_specs=pl.BlockSpec(memory_space=pl.ANY),
        out_shape=jax.ShapeDtypeStruct((Bx, Ox, Lx, 2 * DM), _f32),
        scratch_shapes=[pltpu.VMEM((OP, OP), _bf16),
                        pltpu.VMEM((2 * OP, DM), _bf16),
                        pltpu.VMEM((LT, OP, OP), _bf16),
                        pltpu.VMEM((LT, OP, OP), _bf16),
                        pltpu.VMEM((LT, OP), _f32),
                        pltpu.VMEM((2, LT, Ox, Ox), _f32),
                        pltpu.VMEM((2, LT, Ox, 2 * DM), _f32),
                        pltpu.SemaphoreType.DMA((2, LT)),
                        pltpu.SemaphoreType.DMA((2, LT))],
    )(Xv, dis_matrix, W1, b1.reshape(1, DM), W2, b2.reshape(1, DM))
    return jnp.transpose(outv, (0, 2, 1, 3))


# speedup vs baseline: 3.1886x; 1.0791x over previous
import jax
import jax.numpy as jnp
from jax.experimental import pallas as pl
from jax.experimental.pallas import tpu as pltpu

B, L, O, DM = 8, 48, 100, 128
OP = 112
LT = 24
NG = L // LT
NS = B * NG

_f32 = jnp.float32
_bf16 = jnp.bfloat16


def _main_step(xv_ref, dis_ref, w1_ref, b1_ref, w2_ref, b2_ref,
               out_ref, disw_ref, w1s_ref, x16_ref, xt16_ref, tdn_ref,
               xin_ref, sout_ref, in_sem, out_sem):
    b = pl.program_id(0)
    g = pl.program_id(1)
    s = b * NG + g
    slot = jax.lax.rem(s, 2)
    nslot = 1 - slot

    def in_copy(sl, bb, gg, t):
        return pltpu.make_async_copy(
            xv_ref.at[bb, :, gg * LT + t, :], xin_ref.at[sl, t],
            in_sem.at[sl, t])

    def out_copy(sl, bb, gg, t):
        return pltpu.make_async_copy(
            sout_ref.at[sl, t], out_ref.at[bb, :, gg * LT + t, :],
            out_sem.at[sl, t])

    @pl.when(s == 0)
    def _first():
        for t in range(LT):
            in_copy(0, 0, 0, t).start()

    @pl.when(s < NS - 1)
    def _prefetch():
        nb = jnp.where(g == NG - 1, b + 1, b)
        ng = jnp.where(g == NG - 1, 0, g + 1)
        for t in range(LT):
            in_copy(nslot, nb, ng, t).start()

    @pl.when(pl.program_id(0) + pl.program_id(1) == 0)
    def _init():
        dis = dis_ref[...]
        sd = jnp.sqrt(dis)
        dw = jnp.where(dis <= 2.0, sd, 0.0) / jnp.sum(sd, axis=1,
                                                      keepdims=True)
        disw_ref[...] = jnp.zeros((OP, OP), _bf16)
        disw_ref[:O, :O] = dw.astype(_bf16)
        w1 = w1_ref[...]
        w1s_ref[...] = jnp.zeros((2 * OP, DM), _bf16)
        w1s_ref[:O] = w1[:O].astype(_bf16)
        w1s_ref[O:O + 1] = b1_ref[...].astype(_bf16)
        w1s_ref[OP:OP + O] = w1[O:].astype(_bf16)
        x16_ref[...] = jnp.zeros((LT, OP, OP), _bf16)
        x16_ref[:, :, O:O + 1] = jnp.ones((LT, OP, 1), _bf16)
        xt16_ref[...] = jnp.zeros((LT, OP, OP), _bf16)
        tdn_ref[...] = jnp.zeros((LT, OP), _f32)

    for t in range(LT):
        in_copy(slot, b, g, t).wait()

    @pl.when(s >= 2)
    def _drain_prev():
        pb = jnp.where(g == NG - 1, b, jnp.where(g == 0, b - 1, b))
        ps = s - 2
        pbb = ps // NG
        pgg = jax.lax.rem(ps, NG)
        for t in range(LT):
            out_copy(slot, pbb, pgg, t).wait()

    w1s = w1s_ref[...]
    w2 = w2_ref[...].astype(_bf16)
    b2v = b2_ref[0]
    disw = disw_ref[...]

    x3 = xin_ref[slot]
    xts = [x3[t].T for t in range(LT)]
    x16_ref[:, :O, :O] = x3.astype(_bf16)
    for t in range(LT):
        xt16_ref[t, :O, :O] = xts[t].astype(_bf16)

    td = jnp.sum(x3 + jnp.stack(xts), axis=1)
    tdn_ref[:, :O] = td / jnp.sum(td, axis=1, keepdims=True)

    x16_3 = x16_ref[...]
    xt16_3 = xt16_ref[...]
    xall = x16_3.reshape(LT * OP, OP)
    xtall = xt16_3.reshape(LT * OP, OP)
    yall = (jnp.dot(xall, w1s[:OP], preferred_element_type=_f32)
            + jnp.dot(xtall, w1s[OP:], preferred_element_type=_f32))
    y16 = yall.astype(_bf16)

    ycat = jnp.concatenate([y16[t * OP:(t + 1) * OP] for t in range(LT)],
                           axis=1)
    fall = jnp.dot(disw, ycat, preferred_element_type=_f32)

    tdnv = tdn_ref[...]
    geo_in = []
    sem_in = []
    for t in range(LT):
        sl = slice(t * OP, (t + 1) * OP)
        tdn16 = tdnv[t:t + 1].astype(_bf16)
        deg_w = jnp.where((x16_3[t] > 0) | (xt16_3[t] > 0),
                          jnp.broadcast_to(tdn16, (OP, OP)),
                          _bf16(0))
        bt = jnp.dot(deg_w, y16[sl], preferred_element_type=_f32)
        yt = yall[sl]
        geo_in.append((yt + fall[:, t * DM:(t + 1) * DM]).astype(_bf16))
        sem_in.append((yt + bt).astype(_bf16))

    geo_all = jnp.dot(jnp.concatenate(geo_in, axis=0), w2,
                      preferred_element_type=_f32) + b2v
    sem_all = jnp.dot(jnp.concatenate(sem_in, axis=0), w2,
                      preferred_element_type=_f32) + b2v
    for t in range(LT):
        sout_ref[slot, t] = jnp.concatenate(
            [geo_all[t * OP:t * OP + O], sem_all[t * OP:t * OP + O]],
            axis=-1)
        out_copy(slot, b, g, t).start()

    @pl.when(s == NS - 1)
    def _drain_last():
        ps = s - 1
        pbb = ps // NG
        pgg = jax.lax.rem(ps, NG)
        for t in range(LT):
            out_copy(nslot, pbb, pgg, t).wait()
        for t in range(LT):
            out_copy(slot, b, g, t).wait()


def kernel(X, dis_matrix, W1, b1, W2, b2):
    Bx, Lx, Ox, _ = X.shape
    Xv = jnp.transpose(X, (0, 2, 1, 3))

    outv = pl.pallas_call(
        _main_step,
        grid=(Bx, Lx // LT),
        in_specs=[
            pl.BlockSpec(memory_space=pl.ANY),
            pl.BlockSpec((Ox, Ox), lambda b, g: (0, 0)),
            pl.BlockSpec((2 * Ox, DM), lambda b, g: (0, 0)),
            pl.BlockSpec((1, DM), lambda b, g: (0, 0)),
            pl.BlockSpec((DM, DM), lambda b, g: (0, 0)),
            pl.BlockSpec((1, DM), lambda b, g: (0, 0)),
        ],
        out_specs=pl.BlockSpec(memory_space=pl.ANY),
        out_shape=jax.ShapeDtypeStruct((Bx, Ox, Lx, 2 * DM), _f32),
        scratch_shapes=[pltpu.VMEM((OP, OP), _bf16),
                        pltpu.VMEM((2 * OP, DM), _bf16),
                        pltpu.VMEM((LT, OP, OP), _bf16),
                        pltpu.VMEM((LT, OP, OP), _bf16),
                        pltpu.VMEM((LT, OP), _f32),
                        pltpu.VMEM((2, LT, Ox, Ox), _f32),
                        pltpu.VMEM((2, LT, Ox, 2 * DM), _f32),
                        pltpu.SemaphoreType.DMA((2, LT)),
                        pltpu.SemaphoreType.DMA((2, LT))],
    )(Xv, dis_matrix, W1, b1.reshape(1, DM), W2, b2.reshape(1, DM))
    return jnp.transpose(outv, (0, 2, 1, 3))


# speedup vs baseline: 3.4661x; 1.0870x over previous
import jax
import jax.numpy as jnp
from jax.experimental import pallas as pl
from jax.experimental.pallas import tpu as pltpu

B, L, O, DM = 8, 48, 100, 128
OP = 112
LT = 48
NG = L // LT
NS = B * NG

_f32 = jnp.float32
_bf16 = jnp.bfloat16


def _main_step(xv_ref, dis_ref, w1_ref, b1_ref, w2_ref, b2_ref,
               out_ref, disw_ref, w1s_ref, x16_ref, xt16_ref, tdn_ref,
               xin_ref, sout_ref, in_sem, out_sem):
    b = pl.program_id(0)
    g = pl.program_id(1)
    s = b * NG + g
    slot = jax.lax.rem(s, 2)
    nslot = 1 - slot

    def in_copy(sl, bb, gg, t):
        return pltpu.make_async_copy(
            xv_ref.at[bb, :, gg * LT + t, :], xin_ref.at[sl, t],
            in_sem.at[sl, t])

    def out_copy(sl, bb, gg, t):
        return pltpu.make_async_copy(
            sout_ref.at[sl, t], out_ref.at[bb, :, gg * LT + t, :],
            out_sem.at[sl, t])

    @pl.when(s == 0)
    def _first():
        for t in range(LT):
            in_copy(0, 0, 0, t).start()

    @pl.when(s < NS - 1)
    def _prefetch():
        nb = jnp.where(g == NG - 1, b + 1, b)
        ng = jnp.where(g == NG - 1, 0, g + 1)
        for t in range(LT):
            in_copy(nslot, nb, ng, t).start()

    @pl.when(pl.program_id(0) + pl.program_id(1) == 0)
    def _init():
        dis = dis_ref[...]
        sd = jnp.sqrt(dis)
        dw = jnp.where(dis <= 2.0, sd, 0.0) / jnp.sum(sd, axis=1,
                                                      keepdims=True)
        disw_ref[...] = jnp.zeros((OP, OP), _bf16)
        disw_ref[:O, :O] = dw.astype(_bf16)
        w1 = w1_ref[...]
        w1s_ref[...] = jnp.zeros((2 * OP, DM), _bf16)
        w1s_ref[:O] = w1[:O].astype(_bf16)
        w1s_ref[O:O + 1] = b1_ref[...].astype(_bf16)
        w1s_ref[OP:OP + O] = w1[O:].astype(_bf16)
        x16_ref[...] = jnp.zeros((LT, OP, OP), _bf16)
        x16_ref[:, :, O:O + 1] = jnp.ones((LT, OP, 1), _bf16)
        xt16_ref[...] = jnp.zeros((LT, OP, OP), _bf16)
        tdn_ref[...] = jnp.zeros((LT, OP), _f32)

    for t in range(LT):
        in_copy(slot, b, g, t).wait()

    @pl.when(s >= 2)
    def _drain_prev():
        pb = jnp.where(g == NG - 1, b, jnp.where(g == 0, b - 1, b))
        ps = s - 2
        pbb = ps // NG
        pgg = jax.lax.rem(ps, NG)
        for t in range(LT):
            out_copy(slot, pbb, pgg, t).wait()

    w1s = w1s_ref[...]
    w2 = w2_ref[...].astype(_bf16)
    b2v = b2_ref[0]
    disw = disw_ref[...]

    x3 = xin_ref[slot]
    xts = [x3[t].T for t in range(LT)]
    x16_ref[:, :O, :O] = x3.astype(_bf16)
    for t in range(LT):
        xt16_ref[t, :O, :O] = xts[t].astype(_bf16)

    td = jnp.sum(x3 + jnp.stack(xts), axis=1)
    tdn_ref[:, :O] = td / jnp.sum(td, axis=1, keepdims=True)

    x16_3 = x16_ref[...]
    xt16_3 = xt16_ref[...]
    xall = x16_3.reshape(LT * OP, OP)
    xtall = xt16_3.reshape(LT * OP, OP)
    yall = (jnp.dot(xall, w1s[:OP], preferred_element_type=_f32)
            + jnp.dot(xtall, w1s[OP:], preferred_element_type=_f32))
    y16 = yall.astype(_bf16)

    ycat = jnp.concatenate([y16[t * OP:(t + 1) * OP] for t in range(LT)],
                           axis=1)
    fall = jnp.dot(disw, ycat, preferred_element_type=_f32)

    tdnv = tdn_ref[...]
    geo_in = []
    sem_in = []
    for t in range(LT):
        sl = slice(t * OP, (t + 1) * OP)
        tdn16 = tdnv[t:t + 1].astype(_bf16)
        deg_w = jnp.where((x16_3[t] > 0) | (xt16_3[t] > 0),
                          jnp.broadcast_to(tdn16, (OP, OP)),
                          _bf16(0))
        bt = jnp.dot(deg_w, y16[sl], preferred_element_type=_f32)
        yt = yall[sl]
        geo_in.append((yt + fall[:, t * DM:(t + 1) * DM]).astype(_bf16))
        sem_in.append((yt + bt).astype(_bf16))

    geo_all = jnp.dot(jnp.concatenate(geo_in, axis=0), w2,
                      preferred_element_type=_f32) + b2v
    sem_all = jnp.dot(jnp.concatenate(sem_in, axis=0), w2,
                      preferred_element_type=_f32) + b2v
    for t in range(LT):
        sout_ref[slot, t] = jnp.concatenate(
            [geo_all[t * OP:t * OP + O], sem_all[t * OP:t * OP + O]],
            axis=-1)
        out_copy(slot, b, g, t).start()

    @pl.when(s == NS - 1)
    def _drain_last():
        ps = s - 1
        pbb = ps // NG
        pgg = jax.lax.rem(ps, NG)
        for t in range(LT):
            out_copy(nslot, pbb, pgg, t).wait()
        for t in range(LT):
            out_copy(slot, b, g, t).wait()


def kernel(X, dis_matrix, W1, b1, W2, b2):
    Bx, Lx, Ox, _ = X.shape
    Xv = jnp.transpose(X, (0, 2, 1, 3))

    outv = pl.pallas_call(
        _main_step,
        grid=(Bx, Lx // LT),
        in_specs=[
            pl.BlockSpec(memory_space=pl.ANY),
            pl.BlockSpec((Ox, Ox), lambda b, g: (0, 0)),
            pl.BlockSpec((2 * Ox, DM), lambda b, g: (0, 0)),
            pl.BlockSpec((1, DM), lambda b, g: (0, 0)),
            pl.BlockSpec((DM, DM), lambda b, g: (0, 0)),
            pl.BlockSpec((1, DM), lambda b, g: (0, 0)),
        ],
        out_specs=pl.BlockSpec(memory_space=pl.ANY),
        out_shape=jax.ShapeDtypeStruct((Bx, Ox, Lx, 2 * DM), _f32),
        scratch_shapes=[pltpu.VMEM((OP, OP), _bf16),
                        pltpu.VMEM((2 * OP, DM), _bf16),
                        pltpu.VMEM((LT, OP, OP), _bf16),
                        pltpu.VMEM((LT, OP, OP), _bf16),
                        pltpu.VMEM((LT, OP), _f32),
                        pltpu.VMEM((2, LT, Ox, Ox), _f32),
                        pltpu.VMEM((2, LT, Ox, 2 * DM), _f32),
                        pltpu.SemaphoreType.DMA((2, LT)),
                        pltpu.SemaphoreType.DMA((2, LT))],
    )(Xv, dis_matrix, W1, b1.reshape(1, DM), W2, b2.reshape(1, DM))
    return jnp.transpose(outv, (0, 2, 1, 3))
